# Initial kernel scaffold; baseline (speedup 1.0000x reference)
import jax
import jax.numpy as jnp
from jax.experimental import pallas as pl


def kernel(x_in, x_passive, log_density, W1, b1, W2, b2):
    raise NotImplementedError("write your pallas kernel here")


# speedup vs baseline: 14.0961x; 14.0961x over previous
from math import pi

import jax
import jax.numpy as jnp
from jax.experimental import pallas as pl
from jax.experimental.pallas import tpu as pltpu

EPS = 1e-06
SCALE = pi


def kernel(x_in, x_passive, log_density, W1, b1, W2, b2):
    B, SO = x_in.shape
    n_knots = W2.shape[1] // SO
    NS = (n_knots - 1) // 2
    HID = W1.shape[1]

    W2r = W2.reshape(HID, SO, n_knots).transpose(0, 2, 1).reshape(HID, SO * n_knots)
    b2r = b2.reshape(SO, n_knots).T.reshape(1, SO * n_knots)
    b1r = b1.reshape(1, HID)

    n_elem = x_passive.size

    def stats_body(xp_ref, out_ref):
        x = xp_ref[...]
        mean = jnp.sum(x) * (1.0 / n_elem)
        var = jnp.sum((x - mean) ** 2) * (1.0 / (n_elem - 1))
        out_ref[0, 0] = mean
        out_ref[0, 1] = jax.lax.rsqrt(var)

    stats = pl.pallas_call(
        stats_body,
        out_shape=jax.ShapeDtypeStruct((1, 2), jnp.float32),
        out_specs=pl.BlockSpec(memory_space=pltpu.SMEM),
    )(x_passive)

    BB = 256 if B % 256 == 0 else B

    def body(xi_ref, xp_ref, ld_ref, w1_ref, b1_ref, w2_ref, b2_ref, st_ref,
             phi_ref, ld_out_ref):
        mu = st_ref[0, 0]
        inv_sd = st_ref[0, 1]
        x = xi_ref[...] * (1.0 / SCALE)
        xn = (xp_ref[...] - mu) * inv_sd
        hid = jnp.tanh(
            jnp.dot(xn, w1_ref[...], preferred_element_type=jnp.float32)
            + b1_ref[...])
        net = jnp.tanh(
            jnp.dot(hid, w2_ref[...], preferred_element_type=jnp.float32)
            + b2_ref[...])

        def seg(i):
            return net[:, i * SO:(i + 1) * SO]

        def wseg(i):
            return net[:, (n_knots - NS + i) * SO:(n_knots - NS + i + 1) * SO]

        mx = wseg(0)
        for s in range(1, NS):
            mx = jnp.maximum(mx, wseg(s))
        ew = [jnp.exp(wseg(s) - mx) for s in range(NS)]
        tot = ew[0]
        for s in range(1, NS):
            tot = tot + ew[s]
        inv_tot = 1.0 / tot
        w = [e * inv_tot for e in ew]

        eh = [jnp.exp(seg(s)) for s in range(NS + 1)]
        den = w[0] * (eh[0] + eh[1])
        for s in range(1, NS):
            den = den + w[s] * (eh[s] + eh[s + 1])
        inv_den = 2.0 / den
        hn = [e * inv_den for e in eh]

        xk = jnp.full_like(x, -EPS)
        phik = jnp.zeros_like(x)
        wk = jnp.zeros_like(x)
        hk = jnp.zeros_like(x)
        hkp1 = jnp.zeros_like(x)
        xkm1 = jnp.zeros_like(x)
        phikm1 = jnp.zeros_like(x)
        for s in range(NS):
            xk_next = xk + w[s]
            m = ((xk < x) & (x <= xk_next)).astype(jnp.float32)
            wk = wk + m * w[s]
            hk = hk + m * hn[s]
            hkp1 = hkp1 + m * hn[s + 1]
            xkm1 = xkm1 + m * xk
            phikm1 = phikm1 + m * phik
            phik = phik + 0.5 * w[s] * (hn[s] + hn[s + 1])
            xk = xk_next

        alpha = (x - xkm1) / wk
        dh = hkp1 - hk
        phi = phikm1 + alpha * wk * (hk + 0.5 * alpha * dh)
        phi_ref[...] = phi
        ld_out_ref[...] = ld_ref[...] - jnp.sum(
            jnp.log(hk + alpha * dh), axis=1, keepdims=True)

    grid = (B // BB,)
    row_spec = pl.BlockSpec((BB, SO), lambda i: (i, 0))
    full = lambda shape: pl.BlockSpec(shape, lambda i: (0, 0))
    phi, ld_out = pl.pallas_call(
        body,
        grid=grid,
        in_specs=[
            row_spec,
            row_spec,
            pl.BlockSpec((BB, 1), lambda i: (i, 0)),
            full(W1.shape),
            full(b1r.shape),
            full(W2r.shape),
            full(b2r.shape),
            pl.BlockSpec(memory_space=pltpu.SMEM),
        ],
        out_specs=[
            row_spec,
            pl.BlockSpec((BB, 1), lambda i: (i, 0)),
        ],
        out_shape=[
            jax.ShapeDtypeStruct((B, SO), jnp.float32),
            jax.ShapeDtypeStruct((B, 1), jnp.float32),
        ],
    )(x_in, x_passive, log_density, W1, b1r, W2r, b2r, stats)
    return (phi, ld_out)


# speedup vs baseline: 16.8949x; 1.1985x over previous
from math import pi

import jax
import jax.numpy as jnp
from jax.experimental import pallas as pl
from jax.experimental.pallas import tpu as pltpu

EPS = 1e-06
SCALE = pi


def kernel(x_in, x_passive, log_density, W1, b1, W2, b2):
    B, SO = x_in.shape
    n_knots = W2.shape[1] // SO
    NS = (n_knots - 1) // 2
    HID = W1.shape[1]

    W2r = W2.reshape(HID, SO, n_knots).transpose(0, 2, 1).reshape(HID, SO * n_knots)
    b2r = b2.reshape(SO, n_knots).T.reshape(1, SO * n_knots)
    b1r = b1.reshape(1, HID)

    n_elem = x_passive.size

    def stats_body(xp_ref, out_ref):
        x = xp_ref[...]
        mean = jnp.sum(x) * (1.0 / n_elem)
        var = jnp.sum((x - mean) ** 2) * (1.0 / (n_elem - 1))
        out_ref[0, 0] = mean
        out_ref[0, 1] = jax.lax.rsqrt(var)

    stats = pl.pallas_call(
        stats_body,
        out_shape=jax.ShapeDtypeStruct((1, 2), jnp.float32),
        out_specs=pl.BlockSpec(memory_space=pltpu.SMEM),
    )(x_passive)

    BB = 256 if B % 256 == 0 else B

    def body(xi_ref, xp_ref, ld_ref, w1_ref, b1_ref, w2_ref, b2_ref, st_ref,
             phi_ref, ld_out_ref):
        mu = st_ref[0, 0]
        inv_sd = st_ref[0, 1]
        x = xi_ref[...] * (1.0 / SCALE)
        xn = (xp_ref[...] - mu) * inv_sd
        hid = jnp.tanh(
            jnp.dot(xn, w1_ref[...], preferred_element_type=jnp.float32)
            + b1_ref[...])
        net = jnp.tanh(
            jnp.dot(hid, w2_ref[...], preferred_element_type=jnp.float32)
            + b2_ref[...])

        def seg(i):
            return net[:, i * SO:(i + 1) * SO]

        def wseg(i):
            return net[:, (n_knots - NS + i) * SO:(n_knots - NS + i + 1) * SO]

        u = [jnp.exp(wseg(s)) for s in range(NS)]
        e = [jnp.exp(seg(s)) for s in range(NS + 1)]

        cums = [u[0]]
        for s in range(1, NS):
            cums.append(cums[-1] + u[s])
        dparts = [u[0] * (e[0] + e[1])]
        for s in range(1, NS):
            dparts.append(dparts[-1] + u[s] * (e[s] + e[s + 1]))
        U = cums[-1]
        D = dparts[-1]
        xU = x * U

        m = (xU <= cums[0]).astype(jnp.float32)
        uk = m * u[0]
        ek = m * e[0]
        ekp1 = m * e[1]
        Ck = jnp.zeros_like(x)
        Pk = jnp.zeros_like(x)
        for s in range(1, NS):
            m = ((cums[s - 1] < xU) & (xU <= cums[s])).astype(jnp.float32)
            uk = uk + m * u[s]
            ek = ek + m * e[s]
            ekp1 = ekp1 + m * e[s + 1]
            Ck = Ck + m * cums[s - 1]
            Pk = Pk + m * dparts[s - 1]

        alpha = (xU - Ck) / uk
        de = ekp1 - ek
        ade = alpha * de
        t2 = ek + ade
        invD = 1.0 / D
        phi_ref[...] = (Pk + alpha * uk * (ek + t2)) * invD
        lt = jnp.log(t2 * (U * invD))
        ld_out_ref[...] = ld_ref[...] - (
            jnp.sum(lt, axis=1, keepdims=True) + SO * jnp.log(2.0))

    grid = (B // BB,)
    row_spec = pl.BlockSpec((BB, SO), lambda i: (i, 0))
    full = lambda shape: pl.BlockSpec(shape, lambda i: (0, 0))
    phi, ld_out = pl.pallas_call(
        body,
        grid=grid,
        in_specs=[
            row_spec,
            row_spec,
            pl.BlockSpec((BB, 1), lambda i: (i, 0)),
            full(W1.shape),
            full(b1r.shape),
            full(W2r.shape),
            full(b2r.shape),
            pl.BlockSpec(memory_space=pltpu.SMEM),
        ],
        out_specs=[
            row_spec,
            pl.BlockSpec((BB, 1), lambda i: (i, 0)),
        ],
        out_shape=[
            jax.ShapeDtypeStruct((B, SO), jnp.float32),
            jax.ShapeDtypeStruct((B, 1), jnp.float32),
        ],
    )(x_in, x_passive, log_density, W1, b1r, W2r, b2r, stats)
    return (phi, ld_out)
